# Initial kernel scaffold; baseline (speedup 1.0000x reference)
import jax
import jax.numpy as jnp
from jax.experimental import pallas as pl


def kernel(demo_inputs, demo_outputs, test_input, enc_w1, enc_b1, enc_w2, enc_b2, enc_lw, enc_lb, gu_w1, gu_b1, gu_w2, gu_b2, gt_w1, gt_b1, gt_w2, gt_b2, stem_w, stem_b, codebook, dec_w, dec_b):
    raise NotImplementedError("write your pallas kernel here")


# speedup vs baseline: 1.8517x; 1.8517x over previous
import functools

import jax
import jax.numpy as jnp
from jax.experimental import pallas as pl

F32 = jnp.float32


def _pick_bm(m, cap=2048):
    for c in (1792, 2048, 1024, 512, 256, 128, 64, 32, 16, 8):
        if c <= cap and m % c == 0:
            return c
    return m


def _mm_act_body(x_ref, w_ref, b_ref, o_ref, *, act):
    y = jnp.dot(x_ref[...], w_ref[...], preferred_element_type=F32) + b_ref[...]
    if act == "relu":
        y = jnp.maximum(y, 0.0)
    elif act == "sigmoid":
        y = jax.nn.sigmoid(y)
    o_ref[...] = y


def _mm_act(x, w, b, act):
    m, k = x.shape
    n = w.shape[1]
    bm = _pick_bm(m)
    return pl.pallas_call(
        functools.partial(_mm_act_body, act=act),
        grid=(m // bm,),
        in_specs=[
            pl.BlockSpec((bm, k), lambda i: (i, 0)),
            pl.BlockSpec((k, n), lambda i: (0, 0)),
            pl.BlockSpec((1, n), lambda i: (0, 0)),
        ],
        out_specs=pl.BlockSpec((bm, n), lambda i: (i, 0)),
        out_shape=jax.ShapeDtypeStruct((m, n), F32),
    )(x, w, b.reshape(1, n))


def _mm_relu_psum_body(x_ref, w_ref, b_ref, o_ref):
    t = pl.program_id(1)
    y = jnp.dot(x_ref[0], w_ref[...], preferred_element_type=F32) + b_ref[...]
    y = jnp.maximum(y, 0.0)
    part = jnp.sum(y, axis=0, keepdims=True).reshape(1, 1, -1)

    @pl.when(t == 0)
    def _():
        o_ref[...] = jnp.zeros_like(o_ref)

    o_ref[...] += part


def _mm_relu_pool_sum(x, w, b):
    bimg, p, k = x.shape
    n = w.shape[1]
    bm = _pick_bm(p)
    out = pl.pallas_call(
        _mm_relu_psum_body,
        grid=(bimg, p // bm),
        in_specs=[
            pl.BlockSpec((1, bm, k), lambda bi, t: (bi, t, 0)),
            pl.BlockSpec((k, n), lambda bi, t: (0, 0)),
            pl.BlockSpec((1, n), lambda bi, t: (0, 0)),
        ],
        out_specs=pl.BlockSpec((1, 1, n), lambda bi, t: (bi, 0, 0)),
        out_shape=jax.ShapeDtypeStruct((bimg, 1, n), F32),
    )(x, w, b.reshape(1, n))
    return out.reshape(bimg, n)


def _hyper_body(hs_ref, lwt_ref, lb_ref, gu1t_ref, gu1b_ref, gu2t_ref,
                gu2b_ref, gt1t_ref, gt1b_ref, gt2t_ref, gt2b_ref,
                wu_ref, wt_ref, *, inv_pool, inv_b):
    c = pl.program_id(0)
    h = hs_ref[...] * inv_pool
    te = jnp.dot(h, lwt_ref[...], preferred_element_type=F32) + lb_ref[...]
    te = jnp.sum(te, axis=0, keepdims=True) * inv_b
    hu = jnp.maximum(
        jnp.dot(te, gu1t_ref[...], preferred_element_type=F32) + gu1b_ref[...], 0.0)
    wu_ref[...] = (jnp.dot(hu, gu2t_ref[...], preferred_element_type=F32)
                   + gu2b_ref[...])

    @pl.when(c == 0)
    def _():
        ht = jnp.maximum(
            jnp.dot(te, gt1t_ref[...], preferred_element_type=F32) + gt1b_ref[...],
            0.0)
        wt_ref[...] = (jnp.dot(ht, gt2t_ref[...], preferred_element_type=F32)
                       + gt2b_ref[...])


def _hyper(hsum, enc_lw, enc_lb, gu_w1, gu_b1, gu_w2, gu_b2,
           gt_w1, gt_b1, gt_w2, gt_b2, n_pool):
    bimg = hsum.shape[0]
    upd = gu_w2.shape[0]
    tau = gt_w2.shape[0]
    nch = 8
    chunk = upd // nch
    full = lambda shape: pl.BlockSpec(shape, lambda c: tuple(0 for _ in shape))
    wu, wt = pl.pallas_call(
        functools.partial(_hyper_body, inv_pool=1.0 / n_pool, inv_b=1.0 / bimg),
        grid=(nch,),
        in_specs=[
            full(hsum.shape),
            full((32, 128)), full((1, 128)),
            full((128, 128)), full((1, 128)),
            pl.BlockSpec((128, chunk), lambda c: (0, c)),
            pl.BlockSpec((1, chunk), lambda c: (0, c)),
            full((128, 64)), full((1, 64)),
            full((64, tau)), full((1, tau)),
        ],
        out_specs=[
            pl.BlockSpec((1, chunk), lambda c: (0, c)),
            full((1, tau)),
        ],
        out_shape=[
            jax.ShapeDtypeStruct((1, upd), F32),
            jax.ShapeDtypeStruct((1, tau), F32),
        ],
    )(hsum, enc_lw.T, enc_lb.reshape(1, -1),
      gu_w1.T, gu_b1.reshape(1, -1), gu_w2.T, gu_b2.reshape(1, -1),
      gt_w1.T, gt_b1.reshape(1, -1), gt_w2.T, gt_b2.reshape(1, -1))
    return wu.reshape(-1), wt.reshape(-1)


def _step_body(v0_ref, v1_ref, wfull_ref, cbt_ref, cb_ref, o_ref, *, rb, w, ch):
    x0 = v0_ref[0]
    x1 = v1_ref[0]
    xfull = jnp.concatenate([x0, x1[:2]], axis=0)
    pieces = []
    for dy in range(3):
        for dx in range(3):
            pieces.append(xfull[dy:dy + rb, dx:dx + w, :])
    im2 = jnp.concatenate(pieces, axis=-1).reshape(rb * w, 9 * ch)
    p = jnp.dot(im2, wfull_ref[...], preferred_element_type=F32)
    delta = jnp.maximum(p[:, :ch], 0.0)
    beta = jax.nn.sigmoid(p[:, ch:])
    center = xfull[1:1 + rb, 1:1 + w, :].reshape(rb * w, ch)
    z = beta * center + (1.0 - beta) * delta
    cbt = cbt_ref[...]
    dists = (jnp.sum(cbt * cbt, axis=0, keepdims=True)
             - 2.0 * jnp.dot(z, cbt, preferred_element_type=F32))
    idx = jnp.argmin(dists, axis=1)
    onehot = (jax.lax.broadcasted_iota(jnp.int32, dists.shape, 1)
              == idx[:, None]).astype(F32)
    zq = jnp.dot(onehot, cb_ref[...], preferred_element_type=F32)
    o_ref[0] = zq.reshape(rb, w, ch)


def _nca_step(state, wfull, cbt, cb, rb):
    b, h, w, ch = state.shape
    nr = h // rb
    ncodes = cb.shape[0]
    sp = jnp.pad(state, ((0, 0), (1, rb - 1), (1, 1), (0, 0)))
    wp = w + 2
    return pl.pallas_call(
        functools.partial(_step_body, rb=rb, w=w, ch=ch),
        grid=(b, nr),
        in_specs=[
            pl.BlockSpec((1, rb, wp, ch), lambda bi, r: (bi, r, 0, 0)),
            pl.BlockSpec((1, rb, wp, ch), lambda bi, r: (bi, r + 1, 0, 0)),
            pl.BlockSpec((9 * ch, 2 * ch), lambda bi, r: (0, 0)),
            pl.BlockSpec((ch, ncodes), lambda bi, r: (0, 0)),
            pl.BlockSpec((ncodes, ch), lambda bi, r: (0, 0)),
        ],
        out_specs=pl.BlockSpec((1, rb, w, ch), lambda bi, r: (bi, r, 0, 0)),
        out_shape=jax.ShapeDtypeStruct((b, h, w, ch), F32),
    )(sp, sp, wfull, cbt, cb)


def _im2col3x3(x_cl):
    b, h, w, c = x_cl.shape
    xp = jnp.pad(x_cl, ((0, 0), (1, 1), (1, 1), (0, 0)))
    pieces = [xp[:, dy:dy + h, dx:dx + w, :]
              for dy in range(3) for dx in range(3)]
    return jnp.concatenate(pieces, axis=-1)


def _conv_w_mat(w):
    return w.transpose(2, 3, 1, 0).reshape(-1, w.shape[0])


def kernel(demo_inputs, demo_outputs, test_input,
           enc_w1, enc_b1, enc_w2, enc_b2, enc_lw, enc_lb,
           gu_w1, gu_b1, gu_w2, gu_b2,
           gt_w1, gt_b1, gt_w2, gt_b2,
           stem_w, stem_b, codebook, dec_w, dec_b):
    ch = stem_w.shape[0]
    ncodes = codebook.shape[0]
    bt, _, h, w = test_input.shape
    bd = demo_inputs.shape[0]
    n_steps = 5
    rb = 8 if h % 8 == 0 else (4 if h % 4 == 0 else 1)

    pairs = jnp.concatenate([demo_inputs, demo_outputs], axis=1)
    pairs_cl = pairs.transpose(0, 2, 3, 1)
    x1 = _im2col3x3(pairs_cl).reshape(bd * h * w, 18)
    h1 = _mm_act(x1, _conv_w_mat(enc_w1), enc_b1, "relu")
    x2 = _im2col3x3(h1.reshape(bd, h, w, 16)).reshape(bd, h * w, 144)
    hsum = _mm_relu_pool_sum(x2, _conv_w_mat(enc_w2), enc_b2)

    wu_flat, wt_flat = _hyper(hsum, enc_lw, enc_lb, gu_w1, gu_b1, gu_w2,
                              gu_b2, gt_w1, gt_b1, gt_w2, gt_b2, h * w)
    w_update = wu_flat.reshape(ch, ch, 3, 3)
    w_tau = wt_flat.reshape(ch, ch)
    wfull_l = _conv_w_mat(w_update)
    wfull_r = jnp.pad(w_tau.T, ((4 * ch, 4 * ch), (0, 0)))
    wfull = jnp.concatenate([wfull_l, wfull_r], axis=1)

    xs = _im2col3x3(test_input.transpose(0, 2, 3, 1)).reshape(bt * h * w, 9)
    state = _mm_act(xs, _conv_w_mat(stem_w), stem_b, "relu").reshape(bt, h, w, ch)

    cbt = codebook.T
    for _ in range(n_steps):
        state = _nca_step(state, wfull, cbt, codebook, rb)

    dw = jnp.pad(dec_w.reshape(1, ch).T, ((0, 0), (0, 7)))
    db = jnp.pad(dec_b.reshape(1, 1), ((0, 0), (0, 7)))
    out = _mm_act(state.reshape(bt * h * w, ch), dw, db.reshape(-1), "sigmoid")
    return out[:, :1].reshape(bt, h, w, 1).transpose(0, 3, 1, 2)
